# Initial kernel scaffold; baseline (speedup 1.0000x reference)
import jax
import jax.numpy as jnp
from jax.experimental import pallas as pl


def kernel(x):
    raise NotImplementedError("write your pallas kernel here")


# speedup vs baseline: 1.5045x; 1.5045x over previous
import functools

import jax
import jax.numpy as jnp
from jax.experimental import pallas as pl
from jax.experimental.pallas import tpu as pltpu

_B = 2048
_D = 512
_MARGIN = 0.2
_EPS = 1e-6
_BM = 256


def _body(a_ref, p_ref, out_ref, p_bf_scratch):
    i = pl.program_id(0)

    @pl.when(i == 0)
    def _():
        p = p_ref[...]
        p_sq = jnp.sum(p * p, axis=1, keepdims=True)
        p_n = p * jax.lax.rsqrt(jnp.maximum(p_sq, 1e-24))
        p_bf_scratch[...] = p_n.astype(jnp.bfloat16)
        out_ref[...] = jnp.zeros((1, 1), jnp.float32)

    a = a_ref[...]
    a_sq_raw = jnp.sum(a * a, axis=1, keepdims=True)
    a_n = a * jax.lax.rsqrt(jnp.maximum(a_sq_raw, 1e-24))
    a_sq = jnp.sum(a_n * a_n, axis=1, keepdims=True)

    cross = jax.lax.dot_general(
        a_n.astype(jnp.bfloat16), p_bf_scratch[...],
        (((1,), (1,)), ((), ())),
        preferred_element_type=jnp.float32,
    )

    row = i * _BM + jax.lax.broadcasted_iota(jnp.int32, (_BM, _B), 0)
    col = jax.lax.broadcasted_iota(jnp.int32, (_BM, _B), 1)
    masked = jnp.where(row == col, -jnp.inf, cross)
    max_cross = jnp.max(masked, axis=1, keepdims=True)
    neg_sq = jnp.maximum(a_sq + 1.0 - 2.0 * max_cross, 0.0)

    p_blk = p_ref[pl.ds(i * _BM, _BM), :]
    pb_sq = jnp.sum(p_blk * p_blk, axis=1, keepdims=True)
    p_blk_n = p_blk * jax.lax.rsqrt(jnp.maximum(pb_sq, 1e-24))
    diff = a_n - p_blk_n + _EPS
    pos_sq = jnp.sum(diff * diff, axis=1, keepdims=True)

    loss_blk = jnp.sum(jnp.maximum(pos_sq - neg_sq + _MARGIN, 0.0),
                       axis=0, keepdims=True)
    out_ref[...] += loss_blk * (1.0 / _B)


@functools.partial(jax.jit)
def kernel(x):
    x2 = x.reshape(_B, 2 * _D)
    out = pl.pallas_call(
        _body,
        grid=(_B // _BM,),
        in_specs=[
            pl.BlockSpec((_BM, _D), lambda i: (i, 0)),
            pl.BlockSpec((_B, _D), lambda i: (0, 1)),
        ],
        out_specs=pl.BlockSpec((1, 1), lambda i: (0, 0)),
        out_shape=jax.ShapeDtypeStruct((1, 1), jnp.float32),
        scratch_shapes=[pltpu.VMEM((_B, _D), jnp.bfloat16)],
    )(x2, x2)
    return out[0, 0]


# speedup vs baseline: 2.1508x; 1.4296x over previous
import jax
import jax.numpy as jnp
from jax.experimental import pallas as pl
from jax.experimental.pallas import tpu as pltpu

_B = 2048
_D = 512
_MARGIN = 0.2
_EPS = 1e-6
_BM = 256
_NB = _B // _BM


def _body(x_ref, out_ref, a_bf, p_bf, pos_s):
    ph = pl.program_id(0)
    i = pl.program_id(1)

    @pl.when(jnp.logical_and(ph == 0, i == 0))
    def _():
        out_ref[...] = jnp.zeros((1, 1), jnp.float32)

    @pl.when(ph == 0)
    def _():
        a = x_ref[:, 0, :]
        p = x_ref[:, 1, :]
        a_n = a * jax.lax.rsqrt(
            jnp.maximum(jnp.sum(a * a, axis=1, keepdims=True), 1e-24))
        p_n = p * jax.lax.rsqrt(
            jnp.maximum(jnp.sum(p * p, axis=1, keepdims=True), 1e-24))
        a_bf[pl.ds(i * _BM, _BM), :] = a_n.astype(jnp.bfloat16)
        p_bf[pl.ds(i * _BM, _BM), :] = p_n.astype(jnp.bfloat16)
        diff = a_n - p_n + _EPS
        pos_s[pl.ds(i * _BM, _BM), :] = jnp.sum(diff * diff, axis=1,
                                                keepdims=True)

    @pl.when(ph == 1)
    def _():
        a_n = a_bf[pl.ds(i * _BM, _BM), :]
        cross = jax.lax.dot_general(
            a_n, p_bf[...],
            (((1,), (1,)), ((), ())),
            preferred_element_type=jnp.float32,
        )
        row = i * _BM + jax.lax.broadcasted_iota(jnp.int32, (_BM, _B), 0)
        col = jax.lax.broadcasted_iota(jnp.int32, (_BM, _B), 1)
        masked = jnp.where(row == col, -jnp.inf, cross)
        max_cross = jnp.max(masked, axis=1, keepdims=True)
        neg_sq = jnp.maximum(2.0 - 2.0 * max_cross, 0.0)
        pos_sq = pos_s[pl.ds(i * _BM, _BM), :]
        loss_blk = jnp.sum(jnp.maximum(pos_sq - neg_sq + _MARGIN, 0.0),
                           axis=0, keepdims=True)
        out_ref[...] += loss_blk * (1.0 / _B)


def kernel(x):
    out = pl.pallas_call(
        _body,
        grid=(2, _NB),
        in_specs=[
            pl.BlockSpec((_BM, 2, _D), lambda ph, i: (i * (1 - ph) + (_NB - 1) * ph, 0, 0)),
        ],
        out_specs=pl.BlockSpec((1, 1), lambda ph, i: (0, 0)),
        out_shape=jax.ShapeDtypeStruct((1, 1), jnp.float32),
        scratch_shapes=[
            pltpu.VMEM((_B, _D), jnp.bfloat16),
            pltpu.VMEM((_B, _D), jnp.bfloat16),
            pltpu.VMEM((_B, 1), jnp.float32),
        ],
    )(x)
    return out[0, 0]


# speedup vs baseline: 4.8330x; 2.2471x over previous
import jax
import jax.numpy as jnp
from jax.experimental import pallas as pl
from jax.experimental.pallas import tpu as pltpu

_B = 2048
_D = 512
_MARGIN = 0.2
_EPS = 1e-6
_BM = 256
_NB = _B // _BM


def _normalize(v):
    return v * jax.lax.rsqrt(
        jnp.maximum(jnp.sum(v * v, axis=1, keepdims=True), 1e-24))


def _body(x_hbm, out_ref, a_f, p_f, p_bf, sem_a, sem_p):
    i = pl.program_id(0)

    @pl.when(i == 0)
    def _():
        cp_p = pltpu.make_async_copy(x_hbm.at[:, 1, :], p_f, sem_p)
        cp_a = pltpu.make_async_copy(x_hbm.at[:, 0, :], a_f, sem_a)
        cp_p.start()
        cp_a.start()
        cp_p.wait()
        p_bf[...] = _normalize(p_f[...]).astype(jnp.bfloat16)
        cp_a.wait()

    a_n = _normalize(a_f[pl.ds(i * _BM, _BM), :])
    cross = jax.lax.dot_general(
        a_n.astype(jnp.bfloat16), p_bf[...],
        (((1,), (1,)), ((), ())),
        preferred_element_type=jnp.float32,
    )
    row = i * _BM + jax.lax.broadcasted_iota(jnp.int32, (_BM, _B), 0)
    col = jax.lax.broadcasted_iota(jnp.int32, (_BM, _B), 1)
    masked = jnp.where(row == col, -jnp.inf, cross)
    max_cross = jnp.max(masked, axis=1, keepdims=True)
    neg_sq = jnp.maximum(2.0 - 2.0 * max_cross, 0.0)
    p_n = p_bf[pl.ds(i * _BM, _BM), :].astype(jnp.float32)
    diff = a_n - p_n + _EPS
    pos_sq = jnp.sum(diff * diff, axis=1, keepdims=True)
    loss_blk = jnp.sum(jnp.maximum(pos_sq - neg_sq + _MARGIN, 0.0),
                       axis=0, keepdims=True)
    out_ref[...] = jnp.where(i == 0, 0.0, out_ref[...]) + loss_blk * (1.0 / _B)


def kernel(x):
    out = pl.pallas_call(
        _body,
        grid=(_NB,),
        in_specs=[pl.BlockSpec(memory_space=pl.ANY)],
        out_specs=pl.BlockSpec((1, 1), lambda i: (0, 0)),
        out_shape=jax.ShapeDtypeStruct((1, 1), jnp.float32),
        scratch_shapes=[
            pltpu.VMEM((_B, _D), jnp.float32),
            pltpu.VMEM((_B, _D), jnp.float32),
            pltpu.VMEM((_B, _D), jnp.bfloat16),
            pltpu.SemaphoreType.DMA,
            pltpu.SemaphoreType.DMA,
        ],
    )(x)
    return out[0, 0]


# speedup vs baseline: 4.9123x; 1.0164x over previous
import jax
import jax.numpy as jnp
from jax.experimental import pallas as pl
from jax.experimental.pallas import tpu as pltpu

_B = 2048
_D = 512
_MARGIN = 0.2
_BM = 256
_NB = _B // _BM


def _normalize(v):
    return v * jax.lax.rsqrt(
        jnp.maximum(jnp.sum(v * v, axis=1, keepdims=True), 1e-24))


def _a_blk_copy(x_hbm, a_f, sem_a, k):
    return pltpu.make_async_copy(
        x_hbm.at[pl.ds(k * _BM, _BM), 0, :],
        a_f.at[pl.ds(k * _BM, _BM), :],
        sem_a)


def _body(x_hbm, out_ref, a_f, p_f, p_bf, sem_a, sem_p):
    i = pl.program_id(0)

    @pl.when(i == 0)
    def _():
        cp_p = pltpu.make_async_copy(x_hbm.at[:, 1, :], p_f, sem_p)
        cp_p.start()
        for k in range(_NB):
            _a_blk_copy(x_hbm, a_f, sem_a, k).start()
        cp_p.wait()
        p_bf[...] = _normalize(p_f[...]).astype(jnp.bfloat16)

    _a_blk_copy(x_hbm, a_f, sem_a, i).wait()

    a = a_f[pl.ds(i * _BM, _BM), :]
    asq = jnp.sum(a * a, axis=1, keepdims=True)
    rinv = jax.lax.rsqrt(jnp.maximum(asq, 1e-24))
    cross = jax.lax.dot_general(
        a.astype(jnp.bfloat16), p_bf[...],
        (((1,), (1,)), ((), ())),
        preferred_element_type=jnp.float32,
    )
    row = i * _BM + jax.lax.broadcasted_iota(jnp.int32, (_BM, _B), 0)
    col = jax.lax.broadcasted_iota(jnp.int32, (_BM, _B), 1)
    masked = jnp.where(row == col, -jnp.inf, cross)
    mx = jnp.max(masked, axis=1, keepdims=True)
    p_blk = p_bf[pl.ds(i * _BM, _BM), :].astype(jnp.float32)
    diag = jnp.sum(a * p_blk, axis=1, keepdims=True)
    neg_sq = jnp.maximum(2.0 - 2.0 * mx * rinv, 0.0)
    pos_sq = jnp.maximum(2.0 - 2.0 * diag * rinv, 0.0)
    loss_blk = jnp.sum(jnp.maximum(pos_sq - neg_sq + _MARGIN, 0.0),
                       axis=0, keepdims=True)
    out_ref[...] = jnp.where(i == 0, 0.0, out_ref[...]) + loss_blk * (1.0 / _B)


def kernel(x):
    out = pl.pallas_call(
        _body,
        grid=(_NB,),
        in_specs=[pl.BlockSpec(memory_space=pl.ANY)],
        out_specs=pl.BlockSpec((1, 1), lambda i: (0, 0)),
        out_shape=jax.ShapeDtypeStruct((1, 1), jnp.float32),
        scratch_shapes=[
            pltpu.VMEM((_B, _D), jnp.float32),
            pltpu.VMEM((_B, _D), jnp.float32),
            pltpu.VMEM((_B, _D), jnp.bfloat16),
            pltpu.SemaphoreType.DMA,
            pltpu.SemaphoreType.DMA,
        ],
    )(x)
    return out[0, 0]


# speedup vs baseline: 4.9589x; 1.0095x over previous
import jax
import jax.numpy as jnp
from jax.experimental import pallas as pl
from jax.experimental.pallas import tpu as pltpu

_B = 2048
_D = 512
_MARGIN = 0.2
_BM = 256
_NB = _B // _BM


def _normalize(v):
    return v * jax.lax.rsqrt(
        jnp.maximum(jnp.sum(v * v, axis=1, keepdims=True), 1e-24))


def _a_blk_copy(x_hbm, a_f, sem_a, k):
    return pltpu.make_async_copy(
        x_hbm.at[pl.ds(k * _BM, _BM), 0, :],
        a_f.at[pl.ds(k * _BM, _BM), :],
        sem_a)


def _body(x_hbm, out_ref, a_f, p_f, p_bf, sem_a, sem_p):
    cp_p = pltpu.make_async_copy(x_hbm.at[:, 1, :], p_f, sem_p)
    cp_p.start()
    for k in range(_NB):
        _a_blk_copy(x_hbm, a_f, sem_a, k).start()
    cp_p.wait()
    p_bf[...] = _normalize(p_f[...]).astype(jnp.bfloat16)

    acc = jnp.zeros((1, 1), jnp.float32)
    for k in range(_NB):
        _a_blk_copy(x_hbm, a_f, sem_a, k).wait()
        a = a_f[pl.ds(k * _BM, _BM), :]
        asq = jnp.sum(a * a, axis=1, keepdims=True)
        rinv = jax.lax.rsqrt(jnp.maximum(asq, 1e-24))
        cross = jax.lax.dot_general(
            a.astype(jnp.bfloat16), p_bf[...],
            (((1,), (1,)), ((), ())),
            preferred_element_type=jnp.float32,
        )
        row = k * _BM + jax.lax.broadcasted_iota(jnp.int32, (_BM, _B), 0)
        col = jax.lax.broadcasted_iota(jnp.int32, (_BM, _B), 1)
        masked = jnp.where(row == col, -jnp.inf, cross)
        mx = jnp.max(masked, axis=1, keepdims=True)
        p_blk = p_bf[pl.ds(k * _BM, _BM), :].astype(jnp.float32)
        diag = jnp.sum(a * p_blk, axis=1, keepdims=True)
        neg_sq = jnp.maximum(2.0 - 2.0 * mx * rinv, 0.0)
        pos_sq = jnp.maximum(2.0 - 2.0 * diag * rinv, 0.0)
        acc = acc + jnp.sum(jnp.maximum(pos_sq - neg_sq + _MARGIN, 0.0),
                            axis=0, keepdims=True)
    out_ref[...] = acc * (1.0 / _B)


def kernel(x):
    out = pl.pallas_call(
        _body,
        in_specs=[pl.BlockSpec(memory_space=pl.ANY)],
        out_specs=pl.BlockSpec(memory_space=pltpu.VMEM),
        out_shape=jax.ShapeDtypeStruct((1, 1), jnp.float32),
        scratch_shapes=[
            pltpu.VMEM((_B, _D), jnp.float32),
            pltpu.VMEM((_B, _D), jnp.float32),
            pltpu.VMEM((_B, _D), jnp.bfloat16),
            pltpu.SemaphoreType.DMA,
            pltpu.SemaphoreType.DMA,
        ],
    )(x)
    return out[0, 0]
